# baseline (speedup 1.0000x reference)
import functools

import jax
import jax.numpy as jnp
from jax import lax
from jax.experimental import pallas as pl
from jax.experimental.pallas import tpu as pltpu
from jax.experimental.pallas import tpu_sc as plsc

F32 = jnp.float32
I32 = jnp.int32

NC = 2
NS = 16
NW = NC * NS

KI = 80


def _tc_precompute(nf, wcat):
    n = nf.shape[0]
    nb = 5
    bn = n // nb

    def body(nf_ref, w_ref, a2, b2, a1, b1, d):
        y = jnp.dot(nf_ref[...], w_ref[...], preferred_element_type=F32)
        a2[...] = y[:, 0:64]
        b2[...] = y[:, 64:128]
        a1[...] = y[:, 128:192]
        b1[...] = y[:, 192:256]
        d[...] = y[:, 256:320]

    return pl.pallas_call(
        body,
        grid=(nb,),
        in_specs=[
            pl.BlockSpec((bn, 128), lambda i: (i, 0)),
            pl.BlockSpec((128, 320), lambda i: (0, 0)),
        ],
        out_specs=[pl.BlockSpec((bn, 64), lambda i: (i, 0))] * 5,
        out_shape=[jax.ShapeDtypeStruct((n, 64), F32)] * 5,
    )(nf, wcat)


def _tc_mlp_o2i(h0, ef, w1c, b1, w2, b2, w3, b3, w4, b4, w5, b5):
    e = h0.shape[0]
    blk = 2560
    g = e // blk

    def body(h0_ref, ef_ref, w1c_r, b1_r, w2_r, b2_r, w3_r, b3_r, w4_r, b4_r,
             w5_r, b5_r, lo, hi):
        x = h0_ref[...] + jnp.dot(ef_ref[...], w1c_r[...],
                                  preferred_element_type=F32) + b1_r[...]
        x = jnp.maximum(x, 0.0)
        x = jnp.maximum(jnp.dot(x, w2_r[...], preferred_element_type=F32)
                        + b2_r[...], 0.0)
        x = jnp.maximum(jnp.dot(x, w3_r[...], preferred_element_type=F32)
                        + b3_r[...], 0.0)
        x = jnp.maximum(jnp.dot(x, w4_r[...], preferred_element_type=F32)
                        + b4_r[...], 0.0)
        y = jnp.dot(x, w5_r[...], preferred_element_type=F32) + b5_r[...]
        lo[...] = y[:, 0:64]
        hi[...] = y[:, 64:128]

    full = lambda r, c: pl.BlockSpec((r, c), lambda i: (0, 0))
    return pl.pallas_call(
        body,
        grid=(g,),
        in_specs=[
            pl.BlockSpec((blk, 64), lambda i: (i, 0)),
            pl.BlockSpec((blk, 16), lambda i: (i, 0)),
            full(16, 64), full(1, 64),
            full(64, 64), full(1, 64),
            full(64, 64), full(1, 64),
            full(64, 64), full(1, 64),
            full(64, 128), full(1, 128),
        ],
        out_specs=[pl.BlockSpec((blk, 64), lambda i: (i, 0))] * 2,
        out_shape=[jax.ShapeDtypeStruct((e, 64), F32)] * 2,
    )(h0, ef, w1c, b1, w2, b2, w3, b3, w4, b4, w5, b5)


def _tc_mlp_i2o(h0, ef, v1c, c1, v2, c2, v3, c3, v4p, c4p):
    e = h0.shape[0]
    blk = 2560
    g = e // blk

    def body(h0_ref, ef_ref, v1c_r, c1_r, v2_r, c2_r, v3_r, c3_r, v4_r, c4_r,
             f12):
        x = h0_ref[...] + jnp.dot(ef_ref[...], v1c_r[...],
                                  preferred_element_type=F32) + c1_r[...]
        x = jnp.maximum(x, 0.0)
        x = jnp.maximum(jnp.dot(x, v2_r[...], preferred_element_type=F32)
                        + c2_r[...], 0.0)
        x = jnp.maximum(jnp.dot(x, v3_r[...], preferred_element_type=F32)
                        + c3_r[...], 0.0)
        y = jnp.dot(x, v4_r[...], preferred_element_type=F32) + c4_r[...]
        k = jax.nn.sigmoid(y[:, 64:65])
        f12[...] = y[:, 0:64] * k

    full = lambda r, c: pl.BlockSpec((r, c), lambda i: (0, 0))
    return pl.pallas_call(
        body,
        grid=(g,),
        in_specs=[
            pl.BlockSpec((blk, 64), lambda i: (i, 0)),
            pl.BlockSpec((blk, 16), lambda i: (i, 0)),
            full(16, 64), full(1, 64),
            full(64, 64), full(1, 64),
            full(64, 64), full(1, 64),
            full(64, 128), full(1, 128),
        ],
        out_specs=pl.BlockSpec((blk, 64), lambda i: (i, 0)),
        out_shape=jax.ShapeDtypeStruct((e, 64), F32),
    )(h0, ef, v1c, c1, v2, c2, v3, c3, v4p, c4p)


def _tc_reduce_o(dout, nfo1, nfo2, u1b, u1c, d1, u2, d2, u3, d3, u4, d4):
    p = dout.shape[0]

    def body(do_r, n1_r, n2_r, u1b_r, u1c_r, d1_r, u2_r, d2_r, u3_r, d3_r,
             u4_r, d4_r, out):
        x = (do_r[...]
             + jnp.dot(n1_r[...], u1b_r[...], preferred_element_type=F32)
             + jnp.dot(n2_r[...], u1c_r[...], preferred_element_type=F32)
             + d1_r[...])
        x = jnp.maximum(x, 0.0)
        x = jnp.maximum(jnp.dot(x, u2_r[...], preferred_element_type=F32)
                        + d2_r[...], 0.0)
        x = jnp.maximum(jnp.dot(x, u3_r[...], preferred_element_type=F32)
                        + d3_r[...], 0.0)
        out[...] = jnp.dot(x, u4_r[...], preferred_element_type=F32) + d4_r[...]

    return pl.pallas_call(
        body,
        out_shape=jax.ShapeDtypeStruct((p, 128), F32),
    )(dout, nfo1, nfo2, u1b, u1c, d1, u2, d2, u3, d3, u4, d4)


def _sc_gather(a2, b2, a1, b1, d, src_o, dst_o, src_i, dst_i, onodes, n):
    e = src_o.shape[0]
    n64 = a2.shape[1]
    p = onodes.shape[0]
    ew = e // NW
    nit = ew // KI
    mesh = plsc.VectorSubcoreMesh(core_axis_name="c", subcore_axis_name="s", num_cores=NC, num_subcores=NS)
    fill_stride, fill_len = 624, 640
    slot_stride, slot_len = 312, 320

    @functools.partial(
        pl.kernel, mesh=mesh,
        compiler_params=pltpu.CompilerParams(use_tc_tiling_on_sc=False, needs_layout_passes=False),
        out_type=(jax.ShapeDtypeStruct((e, n64), F32),
                  jax.ShapeDtypeStruct((e, n64), F32),
                  jax.ShapeDtypeStruct((p, n64), F32),
                  jax.ShapeDtypeStruct((n,), I32)),
        scratch_types=[
            pltpu.VMEM((KI,), I32), pltpu.VMEM((KI,), I32),
            pltpu.VMEM((KI,), I32), pltpu.VMEM((KI,), I32),
            pltpu.VMEM((KI, n64), F32), pltpu.VMEM((KI, n64), F32),
            pltpu.VMEM((KI, n64), F32), pltpu.VMEM((KI, n64), F32),
            pltpu.VMEM((fill_len,), I32),
            pltpu.SemaphoreType.DMA, pltpu.SemaphoreType.DMA,
            pltpu.SemaphoreType.DMA, pltpu.SemaphoreType.DMA,
            pltpu.SemaphoreType.DMA, pltpu.SemaphoreType.DMA,
            pltpu.SemaphoreType.DMA, pltpu.SemaphoreType.DMA,
            pltpu.SemaphoreType.DMA, pltpu.SemaphoreType.DMA,
        ],
    )
    def k(a2_r, b2_r, a1_r, b1_r, d_r, so_r, do_r, si_r, di_r, on_r,
          h0o_r, h0i_r, dout_r, pos_r, isrc0, isrc1, idst0, idst1,
          bufa0, bufa1, bufb0, bufb1, fbuf,
          ss0, ss1, sd0, sd1, sa0, sa1, sb0, sb1, sw0, sw1):
        c = lax.axis_index("c")
        s = lax.axis_index("s")
        wid = s * NC + c
        base = wid * ew
        isrc = (isrc0, isrc1)
        idst = (idst0, idst1)
        bufa = (bufa0, bufa1)
        bufb = (bufb0, bufb1)
        ss = (ss0, ss1)
        sd = (sd0, sd1)
        sa = (sa0, sa1)
        sb = (sb0, sb1)
        sw = (sw0, sw1)

        def do_type(a_hbm, b_hbm, src_hbm, dst_hbm, out_hbm):
            def fire_idx(r0, b):
                cs = pltpu.async_copy(src_hbm.at[pl.ds(r0, KI)], isrc[b], ss[b])
                cd = pltpu.async_copy(dst_hbm.at[pl.ds(r0, KI)], idst[b], sd[b])
                return cs, cd

            def fire_gather(b, cs, cd):
                cs.wait()
                cd.wait()
                ca = pltpu.async_copy(a_hbm.at[isrc[b]], bufa[b], sa[b])
                cb = pltpu.async_copy(b_hbm.at[idst[b]], bufb[b], sb[b])
                return ca, cb

            def add_store(r0, b, ca, cb):
                ca.wait()
                cb.wait()
                ba = bufa[b]
                bb = bufb[b]

                @pl.loop(0, KI, unroll=4)
                def _(r):
                    for cc in range(n64 // 16):
                        sl = pl.ds(cc * 16, 16)
                        ba[r, sl] = ba[r, sl] + bb[r, sl]

                return pltpu.async_copy(ba, out_hbm.at[pl.ds(r0, KI)], sw[b])

            @pl.loop(0, nit - 1, step=2)
            def _(i):
                r0 = base + i * KI
                i0 = fire_idx(r0, 0)
                i1 = fire_idx(r0 + KI, 1)
                g0 = fire_gather(0, *i0)
                g1 = fire_gather(1, *i1)
                w0 = add_store(r0, 0, *g0)
                w1 = add_store(r0 + KI, 1, *g1)
                w0.wait()
                w1.wait()

            if nit % 2:
                r0 = base + (nit - 1) * KI
                i0 = fire_idx(r0, 0)
                g0 = fire_gather(0, *i0)
                add_store(r0, 0, *g0).wait()

        do_type(a2_r, b2_r, so_r, do_r, h0o_r)
        do_type(a1_r, b1_r, si_r, di_r, h0i_r)

        @pl.loop(0, 2)
        def _(j):
            off = jnp.minimum(wid * 2 * KI + j * KI, p - KI)
            pltpu.sync_copy(on_r.at[pl.ds(off, KI)], isrc0)
            pltpu.async_copy(d_r.at[isrc0], bufa0, sa0).wait()
            pltpu.sync_copy(bufa0, dout_r.at[pl.ds(off, KI)])

        @pl.when(c == 0)
        def _():
            @pl.loop(0, fill_len // 16)
            def _(j):
                fbuf[pl.ds(j * 16, 16)] = jnp.full((16,), -1, I32)

            pltpu.sync_copy(fbuf, pos_r.at[pl.ds(s * fill_stride, fill_len)])
            plsc.subcore_barrier()

            @pl.loop(0, slot_len // KI)
            def _(j):
                off = s * slot_stride + j * KI

                @pl.loop(0, KI // 16)
                def _(q):
                    isrc0[pl.ds(q * 16, 16)] = lax.iota(I32, 16) + (off + q * 16)

                pltpu.sync_copy(on_r.at[pl.ds(off, KI)], idst0)
                pltpu.sync_copy(isrc0, pos_r.at[idst0])

    return k(a2, b2, a1, b1, d, src_o, dst_o, src_i, dst_i, onodes)


def _sc_segsum_efi(efi_lo, efi_hi, dst_o, n):
    e = efi_lo.shape[0]
    et = e // NS
    nit = et // KI
    rows_per_tile = n // NS
    zrows = 125
    mesh = plsc.VectorSubcoreMesh(core_axis_name="c", subcore_axis_name="s", num_cores=NC, num_subcores=NS)

    @functools.partial(
        pl.kernel, mesh=mesh,
        compiler_params=pltpu.CompilerParams(use_tc_tiling_on_sc=False, needs_layout_passes=False),
        out_type=(jax.ShapeDtypeStruct((n, 64), F32),
                  jax.ShapeDtypeStruct((n, 64), F32)),
        scratch_types=[
            pltpu.VMEM_SHARED((n, 64), F32),
            pltpu.VMEM((zrows, 64), F32),
            pltpu.VMEM((KI, 64), F32), pltpu.VMEM((KI, 64), F32),
            pltpu.VMEM((KI,), I32), pltpu.VMEM((KI,), I32),
            pltpu.SemaphoreType.DMA, pltpu.SemaphoreType.DMA,
            pltpu.SemaphoreType.DMA, pltpu.SemaphoreType.DMA,
            pltpu.SemaphoreType.DMA, pltpu.SemaphoreType.DMA,
        ],
    )
    def k(elo_r, ehi_r, do_r, nnlo_r, nnhi_r, acc, zbuf,
          ebuf0, ebuf1, idxb0, idxb1, se0, se1, si0, si1, sc0, sc1):
        c = lax.axis_index("c")
        s = lax.axis_index("s")
        ebuf = (ebuf0, ebuf1)
        idxb = (idxb0, idxb1)
        se = (se0, se1)
        si = (si0, si1)
        sc = (sc0, sc1)

        @pl.loop(0, zrows)
        def _(r):
            for cc in range(4):
                zbuf[r, pl.ds(cc * 16, 16)] = jnp.zeros((16,), F32)

        @pl.loop(0, rows_per_tile // zrows)
        def _(j):
            pltpu.sync_copy(zbuf, acc.at[pl.ds(s * rows_per_tile + j * zrows,
                                               zrows)])
        plsc.subcore_barrier()

        def phase(src_hbm, out_hbm):
            def fire(r0, b):
                ce = pltpu.async_copy(src_hbm.at[pl.ds(r0, KI)], ebuf[b], se[b])
                ci = pltpu.async_copy(do_r.at[pl.ds(r0, KI)], idxb[b], si[b])
                return ce, ci

            def process(b, ce, ci):
                ce.wait()
                ci.wait()
                return pltpu.async_copy(ebuf[b], acc.at[idxb[b]], sc[b],
                                        add=True)

            @pl.loop(0, nit - 1, step=2)
            def _(i):
                r0 = s * et + i * KI
                d0 = fire(r0, 0)
                d1 = fire(r0 + KI, 1)
                w0 = process(0, *d0)
                w1 = process(1, *d1)
                w0.wait()
                w1.wait()

            if nit % 2:
                r0 = s * et + (nit - 1) * KI
                d0 = fire(r0, 0)
                process(0, *d0).wait()

            plsc.subcore_barrier()
            r0 = s * rows_per_tile
            pltpu.sync_copy(acc.at[pl.ds(r0, rows_per_tile)],
                            out_hbm.at[pl.ds(r0, rows_per_tile)])

        @pl.when(c == 0)
        def _():
            phase(elo_r, nnlo_r)

        @pl.when(c == 1)
        def _():
            phase(ehi_r, nnhi_r)

    return k(efi_lo, efi_hi, dst_o)


def _sc_seg_f1f2(f12, dst_i, pos, n, p):
    e = f12.shape[0]
    et = e // NS
    nit = et // KI
    half = p // NC
    sum_rows = 2560
    cmb_stride, cmb_len = 156, 160
    neg = jnp.float32(-jnp.inf)
    mesh = plsc.VectorSubcoreMesh(core_axis_name="c", subcore_axis_name="s", num_cores=NC, num_subcores=NS)

    @functools.partial(
        pl.kernel, mesh=mesh,
        compiler_params=pltpu.CompilerParams(use_tc_tiling_on_sc=False, needs_layout_passes=False),
        out_type=(jax.ShapeDtypeStruct((p, 32), F32),
                  jax.ShapeDtypeStruct((p, 32), F32)),
        scratch_types=[
            pltpu.VMEM((half + 8, 32), F32),
            pltpu.VMEM((n,), I32),
            pltpu.VMEM_SHARED((sum_rows, 32), F32),
            pltpu.VMEM_SHARED((NS, 320, 32), F32),
            pltpu.VMEM((KI, 32), F32), pltpu.VMEM((KI, 32), F32),
            pltpu.VMEM((KI, 32), F32), pltpu.VMEM((KI, 32), F32),
            pltpu.VMEM((KI,), I32), pltpu.VMEM((KI,), I32),
            pltpu.VMEM((KI,), I32), pltpu.VMEM((KI,), I32),
            pltpu.VMEM((20, 32), F32), pltpu.VMEM((20, 32), F32),
            pltpu.SemaphoreType.DMA, pltpu.SemaphoreType.DMA,
            pltpu.SemaphoreType.DMA, pltpu.SemaphoreType.DMA,
            pltpu.SemaphoreType.DMA, pltpu.SemaphoreType.DMA,
            pltpu.SemaphoreType.DMA, pltpu.SemaphoreType.DMA,
        ],
    )
    def k(f12_r, di_r, pos_r, nfo1_r, nfo2_r,
          macc, posb, sacc, stage, f1b0, f1b1, f2b0, f2b1,
          db0, db1, ib0, ib1, cmb, tbuf,
          sf0, sf1, sg0, sg1, sd0, sd1, sc0, sc1):
        c = lax.axis_index("c")
        s = lax.axis_index("s")
        base = c * half
        f1b = (f1b0, f1b1)
        f2b = (f2b0, f2b1)
        db = (db0, db1)
        ib = (ib0, ib1)
        sf = (sf0, sf1)
        sg = (sg0, sg1)
        sd = (sd0, sd1)
        sc_ = (sc0, sc1)

        pltpu.sync_copy(pos_r, posb)

        @pl.loop(0, half)
        def _(r):
            macc[r, pl.ds(0, 16)] = jnp.full((16,), neg, F32)
            macc[r, pl.ds(16, 16)] = jnp.full((16,), neg, F32)

        @pl.loop(0, KI)
        def _(r):
            for cc in range(2):
                f1b0[r, pl.ds(cc * 16, 16)] = jnp.zeros((16,), F32)

        pltpu.sync_copy(f1b0, sacc.at[pl.ds(s * 160, KI)])
        pltpu.sync_copy(f1b0, sacc.at[pl.ds(s * 160 + KI, KI)])
        plsc.subcore_barrier()

        def fire(r0, b):
            cf = pltpu.async_copy(f12_r.at[pl.ds(r0, KI), pl.ds(0, 32)],
                                  f1b[b], sf[b])
            cg = pltpu.async_copy(f12_r.at[pl.ds(r0, KI), pl.ds(32, 32)],
                                  f2b[b], sg[b])
            cd = pltpu.async_copy(di_r.at[pl.ds(r0, KI)], db[b], sd[b])
            return cf, cg, cd

        def process(b, cf, cg, cd):
            cd.wait()

            @pl.loop(0, KI // 16)
            def _(j):
                sl = pl.ds(j * 16, 16)
                lv = plsc.load_gather(posb, [db[b][sl]]) - base
                valid = (lv >= 0) & (lv < half)
                ib[b][sl] = jnp.where(valid, lv, half)

            cf.wait()
            w = pltpu.async_copy(f1b[b], sacc.at[ib[b]], sc_[b], add=True)
            cg.wait()

            @pl.loop(0, KI // 16)
            def _(j):
                idx16 = ib[b][pl.ds(j * 16, 16)]
                for l in range(16):
                    lv = idx16[l]
                    for cc in range(2):
                        sl = pl.ds(cc * 16, 16)
                        macc[lv, sl] = jnp.maximum(
                            macc[lv, sl], f2b[b][j * 16 + l, sl])

            return w

        @pl.loop(0, nit - 1, step=2)
        def _(i):
            r0 = s * et + i * KI
            d0 = fire(r0, 0)
            d1 = fire(r0 + KI, 1)
            w0 = process(0, *d0)
            w1 = process(1, *d1)
            w0.wait()
            w1.wait()

        if nit % 2:
            r0 = s * et + (nit - 1) * KI
            d0 = fire(r0, 0)
            process(0, *d0).wait()

        plsc.subcore_barrier()

        for rnd in range(8):
            rbase = rnd * 312
            rows_limit = min(320, half - rbase)
            pltpu.sync_copy(macc.at[pl.ds(rbase, 320)], stage.at[s])
            plsc.subcore_barrier()
            l0 = jnp.minimum(s * 20, rows_limit - 20)
            pltpu.sync_copy(stage.at[0, pl.ds(l0, 20)], cmb)

            @pl.loop(1, NS)
            def _(t):
                pltpu.sync_copy(stage.at[t, pl.ds(l0, 20)], tbuf)

                @pl.loop(0, 20)
                def _(r):
                    for cc in range(2):
                        sl = pl.ds(cc * 16, 16)
                        cmb[r, sl] = jnp.maximum(cmb[r, sl], tbuf[r, sl])

            @pl.loop(0, 20)
            def _(r):
                for cc in range(2):
                    sl = pl.ds(cc * 16, 16)
                    v = cmb[r, sl]
                    cmb[r, sl] = jnp.where(v == neg, jnp.zeros((16,), F32), v)

            pltpu.sync_copy(cmb, nfo2_r.at[pl.ds(base + rbase + l0, 20)])
            plsc.subcore_barrier()

        r0s = s * cmb_stride
        pltpu.sync_copy(sacc.at[pl.ds(r0s, cmb_len)],
                        nfo1_r.at[pl.ds(base + r0s, cmb_len)])

    return k(f12, dst_i, pos)


def _sc_assemble(nn_lo, nn_hi, new_val, onodes):
    n = nn_lo.shape[0]
    p = new_val.shape[0]
    rows_per_tile = n // NS
    slot_stride = 312

    mesh = plsc.VectorSubcoreMesh(core_axis_name="c", subcore_axis_name="s", num_cores=NC, num_subcores=NS)

    @functools.partial(
        pl.kernel, mesh=mesh,
        compiler_params=pltpu.CompilerParams(use_tc_tiling_on_sc=False, needs_layout_passes=False),
        out_type=jax.ShapeDtypeStruct((n, 128), F32),
        scratch_types=[
            pltpu.VMEM((rows_per_tile, 64), F32),
            pltpu.VMEM((KI,), I32),
            pltpu.VMEM((KI, 128), F32),
        ],
    )
    def k(lo_r, hi_r, nv_r, on_r, out_r, bounce, idxb, vbuf):
        c = lax.axis_index("c")
        s = lax.axis_index("s")

        @pl.when(c == 0)
        def _():
            r0 = s * rows_per_tile
            pltpu.sync_copy(lo_r.at[pl.ds(r0, rows_per_tile)], bounce)
            pltpu.sync_copy(bounce, out_r.at[pl.ds(r0, rows_per_tile),
                                             pl.ds(0, 64)])
            pltpu.sync_copy(hi_r.at[pl.ds(r0, rows_per_tile)], bounce)
            pltpu.sync_copy(bounce, out_r.at[pl.ds(r0, rows_per_tile),
                                             pl.ds(64, 64)])
            plsc.subcore_barrier()

            @pl.loop(0, 4)
            def _(j):
                off = s * slot_stride + j * KI
                pltpu.sync_copy(on_r.at[pl.ds(off, KI)], idxb)
                pltpu.sync_copy(nv_r.at[pl.ds(off, KI)], vbuf)
                pltpu.sync_copy(vbuf, out_r.at[idxb])

    return k(nn_lo, nn_hi, new_val, onodes)


def kernel(nf, ef_out, ef_in, params_msg_i2o, params_reduce_o, params_msg_o2i,
           edge_index_out, edge_index_in, output_nodes):
    n = nf.shape[0]
    p = output_nodes.shape[0]

    (w1, b1), (w2, b2), (w3, b3), (w4, b4), (w5, b5) = params_msg_o2i
    (v1, c1), (v2, c2), (v3, c3), (v4, c4) = params_msg_i2o
    (u1, d1), (u2, d2), (u3, d3), (u4, d4) = params_reduce_o

    wcat = jnp.concatenate(
        [w1[:128], w1[128:256], v1[:128], v1[128:256], u1[:128]], axis=1)
    w1c = w1[256:272]
    v1c = v1[256:272]
    u1b = u1[128:160]
    u1c = u1[160:192]
    v4p = jnp.zeros((64, 128), F32)
    v4p = v4p.at[:, 0:32].set(v4[:, 1:33])
    v4p = v4p.at[:, 32:64].set(v4[:, 33:65])
    v4p = v4p.at[:, 64:65].set(v4[:, 0:1])
    c4p = jnp.zeros((1, 128), F32)
    c4p = c4p.at[0, 0:32].set(c4[1:33])
    c4p = c4p.at[0, 32:64].set(c4[33:65])
    c4p = c4p.at[0, 64].set(c4[0])

    row = lambda x: x.reshape(1, -1)

    src_o, dst_o = edge_index_out[0], edge_index_out[1]
    src_i, dst_i = edge_index_in[0], edge_index_in[1]

    a2, b2v, a1, b1v, dproj = _tc_precompute(nf, wcat)

    h0_o, h0_i, dout, pos = _sc_gather(a2, b2v, a1, b1v, dproj,
                                       src_o, dst_o, src_i, dst_i,
                                       output_nodes, n)

    efi_lo, efi_hi = _tc_mlp_o2i(h0_o, ef_out, w1c, row(b1), w2, row(b2),
                                 w3, row(b3), w4, row(b4), w5, row(b5))

    f12 = _tc_mlp_i2o(h0_i, ef_in, v1c, row(c1), v2, row(c2),
                      v3, row(c3), v4p, c4p)

    nn_lo, nn_hi = _sc_segsum_efi(efi_lo, efi_hi, dst_o, n)

    nfo1, nfo2 = _sc_seg_f1f2(f12, dst_i, pos, n, p)

    new_val = _tc_reduce_o(dout, nfo1, nfo2, u1b, u1c, row(d1), u2, row(d2),
                           u3, row(d3), u4, row(d4))

    return _sc_assemble(nn_lo, nn_hi, new_val, output_nodes)


# speedup vs baseline: 1.2528x; 1.2528x over previous
import functools

import jax
import jax.numpy as jnp
from jax import lax
from jax.experimental import pallas as pl
from jax.experimental.pallas import tpu as pltpu
from jax.experimental.pallas import tpu_sc as plsc

F32 = jnp.float32
I32 = jnp.int32

NC = 2
NS = 16
NW = NC * NS

KI = 80


def _tc_precompute(nf, wcat):
    n = nf.shape[0]
    nb = 5
    bn = n // nb

    def body(nf_ref, w_ref, a2, b2, a1, b1, d):
        y = jnp.dot(nf_ref[...], w_ref[...], preferred_element_type=F32)
        a2[...] = y[:, 0:64]
        b2[...] = y[:, 64:128]
        a1[...] = y[:, 128:192]
        b1[...] = y[:, 192:256]
        d[...] = y[:, 256:320]

    return pl.pallas_call(
        body,
        grid=(nb,),
        in_specs=[
            pl.BlockSpec((bn, 128), lambda i: (i, 0)),
            pl.BlockSpec((128, 320), lambda i: (0, 0)),
        ],
        out_specs=[pl.BlockSpec((bn, 64), lambda i: (i, 0))] * 5,
        out_shape=[jax.ShapeDtypeStruct((n, 64), F32)] * 5,
    )(nf, wcat)


def _tc_mlp_o2i(h0, ef, w1c, b1, w2, b2, w3, b3, w4, b4, w5, b5):
    e = h0.shape[0]
    blk = 2560
    g = e // blk

    def body(h0_ref, ef_ref, w1c_r, b1_r, w2_r, b2_r, w3_r, b3_r, w4_r, b4_r,
             w5_r, b5_r, lo, hi):
        x = h0_ref[...] + jnp.dot(ef_ref[...], w1c_r[...],
                                  preferred_element_type=F32) + b1_r[...]
        x = jnp.maximum(x, 0.0)
        x = jnp.maximum(jnp.dot(x, w2_r[...], preferred_element_type=F32)
                        + b2_r[...], 0.0)
        x = jnp.maximum(jnp.dot(x, w3_r[...], preferred_element_type=F32)
                        + b3_r[...], 0.0)
        x = jnp.maximum(jnp.dot(x, w4_r[...], preferred_element_type=F32)
                        + b4_r[...], 0.0)
        y = jnp.dot(x, w5_r[...], preferred_element_type=F32) + b5_r[...]
        lo[...] = y[:, 0:64]
        hi[...] = y[:, 64:128]

    full = lambda r, c: pl.BlockSpec((r, c), lambda i: (0, 0))
    return pl.pallas_call(
        body,
        grid=(g,),
        in_specs=[
            pl.BlockSpec((blk, 64), lambda i: (i, 0)),
            pl.BlockSpec((blk, 16), lambda i: (i, 0)),
            full(16, 64), full(1, 64),
            full(64, 64), full(1, 64),
            full(64, 64), full(1, 64),
            full(64, 64), full(1, 64),
            full(64, 128), full(1, 128),
        ],
        out_specs=[pl.BlockSpec((blk, 64), lambda i: (i, 0))] * 2,
        out_shape=[jax.ShapeDtypeStruct((e, 64), F32)] * 2,
    )(h0, ef, w1c, b1, w2, b2, w3, b3, w4, b4, w5, b5)


def _tc_mlp_i2o(h0, ef, v1c, c1, v2, c2, v3, c3, v4p, c4p):
    e = h0.shape[0]
    blk = 2560
    g = e // blk

    def body(h0_ref, ef_ref, v1c_r, c1_r, v2_r, c2_r, v3_r, c3_r, v4_r, c4_r,
             f12):
        x = h0_ref[...] + jnp.dot(ef_ref[...], v1c_r[...],
                                  preferred_element_type=F32) + c1_r[...]
        x = jnp.maximum(x, 0.0)
        x = jnp.maximum(jnp.dot(x, v2_r[...], preferred_element_type=F32)
                        + c2_r[...], 0.0)
        x = jnp.maximum(jnp.dot(x, v3_r[...], preferred_element_type=F32)
                        + c3_r[...], 0.0)
        y = jnp.dot(x, v4_r[...], preferred_element_type=F32) + c4_r[...]
        k = jax.nn.sigmoid(y[:, 64:65])
        f12[...] = y[:, 0:64] * k

    full = lambda r, c: pl.BlockSpec((r, c), lambda i: (0, 0))
    return pl.pallas_call(
        body,
        grid=(g,),
        in_specs=[
            pl.BlockSpec((blk, 64), lambda i: (i, 0)),
            pl.BlockSpec((blk, 16), lambda i: (i, 0)),
            full(16, 64), full(1, 64),
            full(64, 64), full(1, 64),
            full(64, 64), full(1, 64),
            full(64, 128), full(1, 128),
        ],
        out_specs=pl.BlockSpec((blk, 64), lambda i: (i, 0)),
        out_shape=jax.ShapeDtypeStruct((e, 64), F32),
    )(h0, ef, v1c, c1, v2, c2, v3, c3, v4p, c4p)


def _tc_reduce_o(dout, nfo1, nfo2, u1b, u1c, d1, u2, d2, u3, d3, u4, d4):
    p = dout.shape[0]

    def body(do_r, n1_r, n2_r, u1b_r, u1c_r, d1_r, u2_r, d2_r, u3_r, d3_r,
             u4_r, d4_r, out):
        x = (do_r[...]
             + jnp.dot(n1_r[...], u1b_r[...], preferred_element_type=F32)
             + jnp.dot(n2_r[...], u1c_r[...], preferred_element_type=F32)
             + d1_r[...])
        x = jnp.maximum(x, 0.0)
        x = jnp.maximum(jnp.dot(x, u2_r[...], preferred_element_type=F32)
                        + d2_r[...], 0.0)
        x = jnp.maximum(jnp.dot(x, u3_r[...], preferred_element_type=F32)
                        + d3_r[...], 0.0)
        out[...] = jnp.dot(x, u4_r[...], preferred_element_type=F32) + d4_r[...]

    return pl.pallas_call(
        body,
        out_shape=jax.ShapeDtypeStruct((p, 128), F32),
    )(dout, nfo1, nfo2, u1b, u1c, d1, u2, d2, u3, d3, u4, d4)


def _sc_gather(a2, b2, a1, b1, d, src_o, dst_o, src_i, dst_i, onodes, n):
    e = src_o.shape[0]
    n64 = a2.shape[1]
    p = onodes.shape[0]
    ew = e // NW
    nit = ew // KI
    mesh = plsc.VectorSubcoreMesh(core_axis_name="c", subcore_axis_name="s", num_cores=NC, num_subcores=NS)
    fill_stride, fill_len = 624, 640
    slot_stride, slot_len = 312, 320

    @functools.partial(
        pl.kernel, mesh=mesh,
        compiler_params=pltpu.CompilerParams(use_tc_tiling_on_sc=False, needs_layout_passes=False),
        out_type=(jax.ShapeDtypeStruct((e, n64), F32),
                  jax.ShapeDtypeStruct((e, n64), F32),
                  jax.ShapeDtypeStruct((p, n64), F32),
                  jax.ShapeDtypeStruct((n,), I32)),
        scratch_types=[
            pltpu.VMEM((KI,), I32), pltpu.VMEM((KI,), I32),
            pltpu.VMEM((KI,), I32), pltpu.VMEM((KI,), I32),
            pltpu.VMEM((KI, n64), F32), pltpu.VMEM((KI, n64), F32),
            pltpu.VMEM((KI, n64), F32), pltpu.VMEM((KI, n64), F32),
            pltpu.VMEM((fill_len,), I32),
            pltpu.SemaphoreType.DMA, pltpu.SemaphoreType.DMA,
            pltpu.SemaphoreType.DMA, pltpu.SemaphoreType.DMA,
            pltpu.SemaphoreType.DMA, pltpu.SemaphoreType.DMA,
            pltpu.SemaphoreType.DMA, pltpu.SemaphoreType.DMA,
            pltpu.SemaphoreType.DMA, pltpu.SemaphoreType.DMA,
        ],
    )
    def k(a2_r, b2_r, a1_r, b1_r, d_r, so_r, do_r, si_r, di_r, on_r,
          h0o_r, h0i_r, dout_r, pos_r, isrc0, isrc1, idst0, idst1,
          bufa0, bufa1, bufb0, bufb1, fbuf,
          ss0, ss1, sd0, sd1, sa0, sa1, sb0, sb1, sw0, sw1):
        c = lax.axis_index("c")
        s = lax.axis_index("s")
        wid = s * NC + c
        base = wid * ew
        isrc = (isrc0, isrc1)
        idst = (idst0, idst1)
        bufa = (bufa0, bufa1)
        bufb = (bufb0, bufb1)
        ss = (ss0, ss1)
        sd = (sd0, sd1)
        sa = (sa0, sa1)
        sb = (sb0, sb1)
        sw = (sw0, sw1)

        def do_type(a_hbm, b_hbm, src_hbm, dst_hbm, out_hbm):
            def fire(r0, b):
                pltpu.sync_copy(src_hbm.at[pl.ds(r0, KI)], isrc[b])
                pltpu.sync_copy(dst_hbm.at[pl.ds(r0, KI)], idst[b])
                ca = pltpu.async_copy(a_hbm.at[isrc[b]], bufa[b], sa[b])
                cb = pltpu.async_copy(b_hbm.at[idst[b]], bufb[b], sb[b])
                return ca, cb

            def drain(r0, b, ca, cb):
                ca.wait()
                cb.wait()
                ba = bufa[b]
                bb = bufb[b]

                @pl.loop(0, KI)
                def _(r):
                    for cc in range(n64 // 16):
                        sl = pl.ds(cc * 16, 16)
                        ba[r, sl] = ba[r, sl] + bb[r, sl]

                pltpu.sync_copy(ba, out_hbm.at[pl.ds(r0, KI)])

            @pl.loop(0, nit - 1, step=2)
            def _(i):
                r0 = base + i * KI
                d0 = fire(r0, 0)
                d1 = fire(r0 + KI, 1)
                drain(r0, 0, *d0)
                drain(r0 + KI, 1, *d1)

            if nit % 2:
                r0 = base + (nit - 1) * KI
                d0 = fire(r0, 0)
                drain(r0, 0, *d0)

        do_type(a2_r, b2_r, so_r, do_r, h0o_r)
        do_type(a1_r, b1_r, si_r, di_r, h0i_r)

        @pl.loop(0, 2)
        def _(j):
            off = jnp.minimum(wid * 2 * KI + j * KI, p - KI)
            pltpu.sync_copy(on_r.at[pl.ds(off, KI)], isrc0)
            pltpu.async_copy(d_r.at[isrc0], bufa0, sa0).wait()
            pltpu.sync_copy(bufa0, dout_r.at[pl.ds(off, KI)])

        @pl.when(c == 0)
        def _():
            @pl.loop(0, fill_len // 16)
            def _(j):
                fbuf[pl.ds(j * 16, 16)] = jnp.full((16,), -1, I32)

            pltpu.sync_copy(fbuf, pos_r.at[pl.ds(s * fill_stride, fill_len)])
            plsc.subcore_barrier()

            @pl.loop(0, slot_len // KI)
            def _(j):
                off = s * slot_stride + j * KI

                @pl.loop(0, KI // 16)
                def _(q):
                    isrc0[pl.ds(q * 16, 16)] = lax.iota(I32, 16) + (off + q * 16)

                pltpu.sync_copy(on_r.at[pl.ds(off, KI)], idst0)
                pltpu.sync_copy(isrc0, pos_r.at[idst0])

    return k(a2, b2, a1, b1, d, src_o, dst_o, src_i, dst_i, onodes)


def _sc_segsum_efi(efi_lo, efi_hi, dst_o, n):
    e = efi_lo.shape[0]
    et = e // NS
    nit = et // KI
    rows_per_tile = n // NS
    zrows = 125
    mesh = plsc.VectorSubcoreMesh(core_axis_name="c", subcore_axis_name="s", num_cores=NC, num_subcores=NS)

    @functools.partial(
        pl.kernel, mesh=mesh,
        compiler_params=pltpu.CompilerParams(use_tc_tiling_on_sc=False, needs_layout_passes=False),
        out_type=(jax.ShapeDtypeStruct((n, 64), F32),
                  jax.ShapeDtypeStruct((n, 64), F32)),
        scratch_types=[
            pltpu.VMEM_SHARED((n, 64), F32),
            pltpu.VMEM((zrows, 64), F32),
            pltpu.VMEM((KI, 64), F32), pltpu.VMEM((KI, 64), F32),
            pltpu.VMEM((KI,), I32), pltpu.VMEM((KI,), I32),
            pltpu.SemaphoreType.DMA, pltpu.SemaphoreType.DMA,
            pltpu.SemaphoreType.DMA, pltpu.SemaphoreType.DMA,
            pltpu.SemaphoreType.DMA, pltpu.SemaphoreType.DMA,
        ],
    )
    def k(elo_r, ehi_r, do_r, nnlo_r, nnhi_r, acc, zbuf,
          ebuf0, ebuf1, idxb0, idxb1, se0, se1, si0, si1, sc0, sc1):
        c = lax.axis_index("c")
        s = lax.axis_index("s")
        ebuf = (ebuf0, ebuf1)
        idxb = (idxb0, idxb1)
        se = (se0, se1)
        si = (si0, si1)
        sc = (sc0, sc1)

        @pl.loop(0, zrows)
        def _(r):
            for cc in range(4):
                zbuf[r, pl.ds(cc * 16, 16)] = jnp.zeros((16,), F32)

        @pl.loop(0, rows_per_tile // zrows)
        def _(j):
            pltpu.sync_copy(zbuf, acc.at[pl.ds(s * rows_per_tile + j * zrows,
                                               zrows)])
        plsc.subcore_barrier()

        def phase(src_hbm, out_hbm):
            def fire(r0, b):
                ce = pltpu.async_copy(src_hbm.at[pl.ds(r0, KI)], ebuf[b], se[b])
                ci = pltpu.async_copy(do_r.at[pl.ds(r0, KI)], idxb[b], si[b])
                return ce, ci

            def process(b, ce, ci):
                ce.wait()
                ci.wait()
                return pltpu.async_copy(ebuf[b], acc.at[idxb[b]], sc[b],
                                        add=True)

            @pl.loop(0, nit - 1, step=2)
            def _(i):
                r0 = s * et + i * KI
                d0 = fire(r0, 0)
                d1 = fire(r0 + KI, 1)
                w0 = process(0, *d0)
                w1 = process(1, *d1)
                w0.wait()
                w1.wait()

            if nit % 2:
                r0 = s * et + (nit - 1) * KI
                d0 = fire(r0, 0)
                process(0, *d0).wait()

            plsc.subcore_barrier()
            r0 = s * rows_per_tile
            pltpu.sync_copy(acc.at[pl.ds(r0, rows_per_tile)],
                            out_hbm.at[pl.ds(r0, rows_per_tile)])

        @pl.when(c == 0)
        def _():
            phase(elo_r, nnlo_r)

        @pl.when(c == 1)
        def _():
            phase(ehi_r, nnhi_r)

    return k(efi_lo, efi_hi, dst_o)


def _sc_seg_f1f2(f12, dst_i, pos, n, p):
    e = f12.shape[0]
    et = e // NS
    nit = et // KI
    half = p // NC
    sum_rows = 2560
    cmb_stride, cmb_len = 156, 160
    neg = jnp.float32(-jnp.inf)
    mesh = plsc.VectorSubcoreMesh(core_axis_name="c", subcore_axis_name="s", num_cores=NC, num_subcores=NS)

    @functools.partial(
        pl.kernel, mesh=mesh,
        compiler_params=pltpu.CompilerParams(use_tc_tiling_on_sc=False, needs_layout_passes=False),
        out_type=(jax.ShapeDtypeStruct((p, 32), F32),
                  jax.ShapeDtypeStruct((p, 32), F32)),
        scratch_types=[
            pltpu.VMEM((half + 8, 32), F32),
            pltpu.VMEM((n,), I32),
            pltpu.VMEM_SHARED((sum_rows, 32), F32),
            pltpu.VMEM_SHARED((NS, 320, 32), F32),
            pltpu.VMEM((KI, 32), F32), pltpu.VMEM((KI, 32), F32),
            pltpu.VMEM((KI, 32), F32), pltpu.VMEM((KI, 32), F32),
            pltpu.VMEM((KI,), I32), pltpu.VMEM((KI,), I32),
            pltpu.VMEM((KI,), I32), pltpu.VMEM((KI,), I32),
            pltpu.VMEM((20, 32), F32), pltpu.VMEM((20, 32), F32),
            pltpu.SemaphoreType.DMA, pltpu.SemaphoreType.DMA,
            pltpu.SemaphoreType.DMA, pltpu.SemaphoreType.DMA,
            pltpu.SemaphoreType.DMA, pltpu.SemaphoreType.DMA,
            pltpu.SemaphoreType.DMA, pltpu.SemaphoreType.DMA,
        ],
    )
    def k(f12_r, di_r, pos_r, nfo1_r, nfo2_r,
          macc, posb, sacc, stage, f1b0, f1b1, f2b0, f2b1,
          db0, db1, ib0, ib1, cmb, tbuf,
          sf0, sf1, sg0, sg1, sd0, sd1, sc0, sc1):
        c = lax.axis_index("c")
        s = lax.axis_index("s")
        base = c * half
        f1b = (f1b0, f1b1)
        f2b = (f2b0, f2b1)
        db = (db0, db1)
        ib = (ib0, ib1)
        sf = (sf0, sf1)
        sg = (sg0, sg1)
        sd = (sd0, sd1)
        sc_ = (sc0, sc1)

        pltpu.sync_copy(pos_r, posb)

        @pl.loop(0, half)
        def _(r):
            macc[r, pl.ds(0, 16)] = jnp.full((16,), neg, F32)
            macc[r, pl.ds(16, 16)] = jnp.full((16,), neg, F32)

        @pl.loop(0, KI)
        def _(r):
            for cc in range(2):
                f1b0[r, pl.ds(cc * 16, 16)] = jnp.zeros((16,), F32)

        pltpu.sync_copy(f1b0, sacc.at[pl.ds(s * 160, KI)])
        pltpu.sync_copy(f1b0, sacc.at[pl.ds(s * 160 + KI, KI)])
        plsc.subcore_barrier()

        def fire(r0, b):
            cf = pltpu.async_copy(f12_r.at[pl.ds(r0, KI), pl.ds(0, 32)],
                                  f1b[b], sf[b])
            cg = pltpu.async_copy(f12_r.at[pl.ds(r0, KI), pl.ds(32, 32)],
                                  f2b[b], sg[b])
            cd = pltpu.async_copy(di_r.at[pl.ds(r0, KI)], db[b], sd[b])
            return cf, cg, cd

        def process(b, cf, cg, cd):
            cd.wait()

            @pl.loop(0, KI // 16)
            def _(j):
                sl = pl.ds(j * 16, 16)
                lv = plsc.load_gather(posb, [db[b][sl]]) - base
                valid = (lv >= 0) & (lv < half)
                ib[b][sl] = jnp.where(valid, lv, half)

            cf.wait()
            w = pltpu.async_copy(f1b[b], sacc.at[ib[b]], sc_[b], add=True)
            cg.wait()

            @pl.loop(0, KI // 16)
            def _(j):
                idx16 = ib[b][pl.ds(j * 16, 16)]
                for l in range(16):
                    lv = idx16[l]
                    for cc in range(2):
                        sl = pl.ds(cc * 16, 16)
                        macc[lv, sl] = jnp.maximum(
                            macc[lv, sl], f2b[b][j * 16 + l, sl])

            return w

        @pl.loop(0, nit - 1, step=2)
        def _(i):
            r0 = s * et + i * KI
            d0 = fire(r0, 0)
            d1 = fire(r0 + KI, 1)
            w0 = process(0, *d0)
            w1 = process(1, *d1)
            w0.wait()
            w1.wait()

        if nit % 2:
            r0 = s * et + (nit - 1) * KI
            d0 = fire(r0, 0)
            process(0, *d0).wait()

        plsc.subcore_barrier()

        for rnd in range(8):
            rbase = rnd * 312
            rows_limit = min(320, half - rbase)
            pltpu.sync_copy(macc.at[pl.ds(rbase, 320)], stage.at[s])
            plsc.subcore_barrier()
            l0 = jnp.minimum(s * 20, rows_limit - 20)
            pltpu.sync_copy(stage.at[0, pl.ds(l0, 20)], cmb)

            @pl.loop(1, NS)
            def _(t):
                pltpu.sync_copy(stage.at[t, pl.ds(l0, 20)], tbuf)

                @pl.loop(0, 20)
                def _(r):
                    for cc in range(2):
                        sl = pl.ds(cc * 16, 16)
                        cmb[r, sl] = jnp.maximum(cmb[r, sl], tbuf[r, sl])

            @pl.loop(0, 20)
            def _(r):
                for cc in range(2):
                    sl = pl.ds(cc * 16, 16)
                    v = cmb[r, sl]
                    cmb[r, sl] = jnp.where(v == neg, jnp.zeros((16,), F32), v)

            pltpu.sync_copy(cmb, nfo2_r.at[pl.ds(base + rbase + l0, 20)])
            plsc.subcore_barrier()

        r0s = s * cmb_stride
        pltpu.sync_copy(sacc.at[pl.ds(r0s, cmb_len)],
                        nfo1_r.at[pl.ds(base + r0s, cmb_len)])

    return k(f12, dst_i, pos)


def _sc_assemble(nn_lo, nn_hi, new_val, onodes):
    n = nn_lo.shape[0]
    p = new_val.shape[0]
    rows_per_tile = n // NS
    slot_stride = 312

    mesh = plsc.VectorSubcoreMesh(core_axis_name="c", subcore_axis_name="s", num_cores=NC, num_subcores=NS)

    @functools.partial(
        pl.kernel, mesh=mesh,
        compiler_params=pltpu.CompilerParams(use_tc_tiling_on_sc=False, needs_layout_passes=False),
        out_type=jax.ShapeDtypeStruct((n, 128), F32),
        scratch_types=[
            pltpu.VMEM((rows_per_tile, 64), F32),
            pltpu.VMEM((KI,), I32),
            pltpu.VMEM((KI, 128), F32),
        ],
    )
    def k(lo_r, hi_r, nv_r, on_r, out_r, bounce, idxb, vbuf):
        c = lax.axis_index("c")
        s = lax.axis_index("s")

        @pl.when(c == 0)
        def _():
            r0 = s * rows_per_tile
            pltpu.sync_copy(lo_r.at[pl.ds(r0, rows_per_tile)], bounce)
            pltpu.sync_copy(bounce, out_r.at[pl.ds(r0, rows_per_tile),
                                             pl.ds(0, 64)])
            pltpu.sync_copy(hi_r.at[pl.ds(r0, rows_per_tile)], bounce)
            pltpu.sync_copy(bounce, out_r.at[pl.ds(r0, rows_per_tile),
                                             pl.ds(64, 64)])
            plsc.subcore_barrier()

            @pl.loop(0, 4)
            def _(j):
                off = s * slot_stride + j * KI
                pltpu.sync_copy(on_r.at[pl.ds(off, KI)], idxb)
                pltpu.sync_copy(nv_r.at[pl.ds(off, KI)], vbuf)
                pltpu.sync_copy(vbuf, out_r.at[idxb])

    return k(nn_lo, nn_hi, new_val, onodes)


def kernel(nf, ef_out, ef_in, params_msg_i2o, params_reduce_o, params_msg_o2i,
           edge_index_out, edge_index_in, output_nodes):
    n = nf.shape[0]
    p = output_nodes.shape[0]

    (w1, b1), (w2, b2), (w3, b3), (w4, b4), (w5, b5) = params_msg_o2i
    (v1, c1), (v2, c2), (v3, c3), (v4, c4) = params_msg_i2o
    (u1, d1), (u2, d2), (u3, d3), (u4, d4) = params_reduce_o

    wcat = jnp.concatenate(
        [w1[:128], w1[128:256], v1[:128], v1[128:256], u1[:128]], axis=1)
    w1c = w1[256:272]
    v1c = v1[256:272]
    u1b = u1[128:160]
    u1c = u1[160:192]
    v4p = jnp.zeros((64, 128), F32)
    v4p = v4p.at[:, 0:32].set(v4[:, 1:33])
    v4p = v4p.at[:, 32:64].set(v4[:, 33:65])
    v4p = v4p.at[:, 64:65].set(v4[:, 0:1])
    c4p = jnp.zeros((1, 128), F32)
    c4p = c4p.at[0, 0:32].set(c4[1:33])
    c4p = c4p.at[0, 32:64].set(c4[33:65])
    c4p = c4p.at[0, 64].set(c4[0])

    row = lambda x: x.reshape(1, -1)

    src_o, dst_o = edge_index_out[0], edge_index_out[1]
    src_i, dst_i = edge_index_in[0], edge_index_in[1]

    a2, b2v, a1, b1v, dproj = _tc_precompute(nf, wcat)

    h0_o, h0_i, dout, pos = _sc_gather(a2, b2v, a1, b1v, dproj,
                                       src_o, dst_o, src_i, dst_i,
                                       output_nodes, n)

    efi_lo, efi_hi = _tc_mlp_o2i(h0_o, ef_out, w1c, row(b1), w2, row(b2),
                                 w3, row(b3), w4, row(b4), w5, row(b5))

    f12 = _tc_mlp_i2o(h0_i, ef_in, v1c, row(c1), v2, row(c2),
                      v3, row(c3), v4p, c4p)

    nn_lo, nn_hi = _sc_segsum_efi(efi_lo, efi_hi, dst_o, n)

    nfo1, nfo2 = _sc_seg_f1f2(f12, dst_i, pos, n, p)

    new_val = _tc_reduce_o(dout, nfo1, nfo2, u1b, u1c, row(d1), u2, row(d2),
                           u3, row(d3), u4, row(d4))

    return _sc_assemble(nn_lo, nn_hi, new_val, output_nodes)


# speedup vs baseline: 1.2876x; 1.0277x over previous
import functools

import jax
import jax.numpy as jnp
from jax import lax
from jax.experimental import pallas as pl
from jax.experimental.pallas import tpu as pltpu
from jax.experimental.pallas import tpu_sc as plsc

F32 = jnp.float32
I32 = jnp.int32

NC = 2
NS = 16
NW = NC * NS

KI = 80


def _tc_precompute(nf, wcat):
    n = nf.shape[0]
    nb = 5
    bn = n // nb

    def body(nf_ref, w_ref, a2, b2, a1, b1, d):
        y = jnp.dot(nf_ref[...], w_ref[...], preferred_element_type=F32)
        a2[...] = y[:, 0:64]
        b2[...] = y[:, 64:128]
        a1[...] = y[:, 128:192]
        b1[...] = y[:, 192:256]
        d[...] = y[:, 256:320]

    return pl.pallas_call(
        body,
        grid=(nb,),
        in_specs=[
            pl.BlockSpec((bn, 128), lambda i: (i, 0)),
            pl.BlockSpec((128, 320), lambda i: (0, 0)),
        ],
        out_specs=[pl.BlockSpec((bn, 64), lambda i: (i, 0))] * 5,
        out_shape=[jax.ShapeDtypeStruct((n, 64), F32)] * 5,
    )(nf, wcat)


def _tc_mlp_edges(h0o, ef_o, h0i, ef_i,
                  w1c, b1, w2, b2, w3, b3, w4, b4, w5, b5,
                  v1c, c1, v2, c2, v3, c3, v4p, c4p):
    e = h0o.shape[0]
    blk = 2560
    g = e // blk

    def body(h0o_ref, efo_ref, h0i_ref, efi_ref,
             w1c_r, b1_r, w2_r, b2_r, w3_r, b3_r, w4_r, b4_r, w5_r, b5_r,
             v1c_r, c1_r, v2_r, c2_r, v3_r, c3_r, v4_r, c4_r,
             lo, hi, f12):
        x = h0o_ref[...] + jnp.dot(efo_ref[...], w1c_r[...],
                                   preferred_element_type=F32) + b1_r[...]
        x = jnp.maximum(x, 0.0)
        x = jnp.maximum(jnp.dot(x, w2_r[...], preferred_element_type=F32)
                        + b2_r[...], 0.0)
        x = jnp.maximum(jnp.dot(x, w3_r[...], preferred_element_type=F32)
                        + b3_r[...], 0.0)
        x = jnp.maximum(jnp.dot(x, w4_r[...], preferred_element_type=F32)
                        + b4_r[...], 0.0)
        y = jnp.dot(x, w5_r[...], preferred_element_type=F32) + b5_r[...]
        lo[...] = y[:, 0:64]
        hi[...] = y[:, 64:128]

        x = h0i_ref[...] + jnp.dot(efi_ref[...], v1c_r[...],
                                   preferred_element_type=F32) + c1_r[...]
        x = jnp.maximum(x, 0.0)
        x = jnp.maximum(jnp.dot(x, v2_r[...], preferred_element_type=F32)
                        + c2_r[...], 0.0)
        x = jnp.maximum(jnp.dot(x, v3_r[...], preferred_element_type=F32)
                        + c3_r[...], 0.0)
        y = jnp.dot(x, v4_r[...], preferred_element_type=F32) + c4_r[...]
        k = jax.nn.sigmoid(y[:, 64:65])
        f12[...] = y[:, 0:64] * k

    full = lambda r, c: pl.BlockSpec((r, c), lambda i: (0, 0))
    return pl.pallas_call(
        body,
        grid=(g,),
        in_specs=[
            pl.BlockSpec((blk, 64), lambda i: (i, 0)),
            pl.BlockSpec((blk, 16), lambda i: (i, 0)),
            pl.BlockSpec((blk, 64), lambda i: (i, 0)),
            pl.BlockSpec((blk, 16), lambda i: (i, 0)),
            full(16, 64), full(1, 64),
            full(64, 64), full(1, 64),
            full(64, 64), full(1, 64),
            full(64, 64), full(1, 64),
            full(64, 128), full(1, 128),
            full(16, 64), full(1, 64),
            full(64, 64), full(1, 64),
            full(64, 64), full(1, 64),
            full(64, 128), full(1, 128),
        ],
        out_specs=[pl.BlockSpec((blk, 64), lambda i: (i, 0))] * 3,
        out_shape=[jax.ShapeDtypeStruct((e, 64), F32)] * 3,
    )(h0o, ef_o, h0i, ef_i, w1c, b1, w2, b2, w3, b3, w4, b4, w5, b5,
      v1c, c1, v2, c2, v3, c3, v4p, c4p)


def _tc_reduce_o(dout, nfo1, nfo2, u1b, u1c, d1, u2, d2, u3, d3, u4, d4):
    p = dout.shape[0]

    def body(do_r, n1_r, n2_r, u1b_r, u1c_r, d1_r, u2_r, d2_r, u3_r, d3_r,
             u4_r, d4_r, out):
        x = (do_r[...]
             + jnp.dot(n1_r[...], u1b_r[...], preferred_element_type=F32)
             + jnp.dot(n2_r[...], u1c_r[...], preferred_element_type=F32)
             + d1_r[...])
        x = jnp.maximum(x, 0.0)
        x = jnp.maximum(jnp.dot(x, u2_r[...], preferred_element_type=F32)
                        + d2_r[...], 0.0)
        x = jnp.maximum(jnp.dot(x, u3_r[...], preferred_element_type=F32)
                        + d3_r[...], 0.0)
        out[...] = jnp.dot(x, u4_r[...], preferred_element_type=F32) + d4_r[...]

    return pl.pallas_call(
        body,
        out_shape=jax.ShapeDtypeStruct((p, 128), F32),
    )(dout, nfo1, nfo2, u1b, u1c, d1, u2, d2, u3, d3, u4, d4)


def _sc_gather(a2, b2, a1, b1, d, src_o, dst_o, src_i, dst_i, onodes, n):
    e = src_o.shape[0]
    n64 = a2.shape[1]
    p = onodes.shape[0]
    ew = e // NW
    nit = ew // KI
    mesh = plsc.VectorSubcoreMesh(core_axis_name="c", subcore_axis_name="s", num_cores=NC, num_subcores=NS)
    fill_stride, fill_len = 624, 640
    slot_stride, slot_len = 312, 320

    @functools.partial(
        pl.kernel, mesh=mesh,
        compiler_params=pltpu.CompilerParams(use_tc_tiling_on_sc=False, needs_layout_passes=False),
        out_type=(jax.ShapeDtypeStruct((e, n64), F32),
                  jax.ShapeDtypeStruct((e, n64), F32),
                  jax.ShapeDtypeStruct((p, n64), F32),
                  jax.ShapeDtypeStruct((n,), I32)),
        scratch_types=[
            pltpu.VMEM((KI,), I32), pltpu.VMEM((KI,), I32),
            pltpu.VMEM((KI,), I32), pltpu.VMEM((KI,), I32),
            pltpu.VMEM((KI, n64), F32), pltpu.VMEM((KI, n64), F32),
            pltpu.VMEM((KI, n64), F32), pltpu.VMEM((KI, n64), F32),
            pltpu.VMEM((fill_len,), I32),
            pltpu.SemaphoreType.DMA, pltpu.SemaphoreType.DMA,
            pltpu.SemaphoreType.DMA, pltpu.SemaphoreType.DMA,
            pltpu.SemaphoreType.DMA, pltpu.SemaphoreType.DMA,
            pltpu.SemaphoreType.DMA, pltpu.SemaphoreType.DMA,
            pltpu.SemaphoreType.DMA, pltpu.SemaphoreType.DMA,
        ],
    )
    def k(a2_r, b2_r, a1_r, b1_r, d_r, so_r, do_r, si_r, di_r, on_r,
          h0o_r, h0i_r, dout_r, pos_r, isrc0, isrc1, idst0, idst1,
          bufa0, bufa1, bufb0, bufb1, fbuf,
          ss0, ss1, sd0, sd1, sa0, sa1, sb0, sb1, sw0, sw1):
        c = lax.axis_index("c")
        s = lax.axis_index("s")
        wid = s * NC + c
        base = wid * ew
        isrc = (isrc0, isrc1)
        idst = (idst0, idst1)
        bufa = (bufa0, bufa1)
        bufb = (bufb0, bufb1)
        ss = (ss0, ss1)
        sd = (sd0, sd1)
        sa = (sa0, sa1)
        sb = (sb0, sb1)
        sw = (sw0, sw1)

        def do_type(a_hbm, b_hbm, src_hbm, dst_hbm, out_hbm):
            def fire(r0, b):
                pltpu.sync_copy(src_hbm.at[pl.ds(r0, KI)], isrc[b])
                pltpu.sync_copy(dst_hbm.at[pl.ds(r0, KI)], idst[b])
                ca = pltpu.async_copy(a_hbm.at[isrc[b]], bufa[b], sa[b])
                cb = pltpu.async_copy(b_hbm.at[idst[b]], bufb[b], sb[b])
                return ca, cb

            def drain(r0, b, ca, cb):
                ca.wait()
                cb.wait()
                ba = bufa[b]
                bb = bufb[b]

                @pl.loop(0, KI)
                def _(r):
                    for cc in range(n64 // 16):
                        sl = pl.ds(cc * 16, 16)
                        ba[r, sl] = ba[r, sl] + bb[r, sl]

                return pltpu.async_copy(ba, out_hbm.at[pl.ds(r0, KI)], sw[b])

            @pl.loop(0, nit - 1, step=2)
            def _(i):
                r0 = base + i * KI
                d0 = fire(r0, 0)
                d1 = fire(r0 + KI, 1)
                w0 = drain(r0, 0, *d0)
                w1 = drain(r0 + KI, 1, *d1)
                w0.wait()
                w1.wait()

            if nit % 2:
                r0 = base + (nit - 1) * KI
                d0 = fire(r0, 0)
                drain(r0, 0, *d0).wait()

        do_type(a2_r, b2_r, so_r, do_r, h0o_r)
        do_type(a1_r, b1_r, si_r, di_r, h0i_r)

        @pl.loop(0, 2)
        def _(j):
            off = jnp.minimum(wid * 2 * KI + j * KI, p - KI)
            pltpu.sync_copy(on_r.at[pl.ds(off, KI)], isrc0)
            pltpu.async_copy(d_r.at[isrc0], bufa0, sa0).wait()
            pltpu.sync_copy(bufa0, dout_r.at[pl.ds(off, KI)])

        @pl.when(c == 0)
        def _():
            @pl.loop(0, fill_len // 16)
            def _(j):
                fbuf[pl.ds(j * 16, 16)] = jnp.full((16,), -1, I32)

            pltpu.sync_copy(fbuf, pos_r.at[pl.ds(s * fill_stride, fill_len)])
            plsc.subcore_barrier()

            @pl.loop(0, slot_len // KI)
            def _(j):
                off = s * slot_stride + j * KI

                @pl.loop(0, KI // 16)
                def _(q):
                    isrc0[pl.ds(q * 16, 16)] = lax.iota(I32, 16) + (off + q * 16)

                pltpu.sync_copy(on_r.at[pl.ds(off, KI)], idst0)
                pltpu.sync_copy(isrc0, pos_r.at[idst0])

    return k(a2, b2, a1, b1, d, src_o, dst_o, src_i, dst_i, onodes)


def _sc_segsum_efi(efi_lo, efi_hi, dst_o, n):
    e = efi_lo.shape[0]
    et = e // NS
    nit = et // KI
    rows_per_tile = n // NS
    zrows = 125
    mesh = plsc.VectorSubcoreMesh(core_axis_name="c", subcore_axis_name="s", num_cores=NC, num_subcores=NS)

    @functools.partial(
        pl.kernel, mesh=mesh,
        compiler_params=pltpu.CompilerParams(use_tc_tiling_on_sc=False, needs_layout_passes=False),
        out_type=(jax.ShapeDtypeStruct((n, 64), F32),
                  jax.ShapeDtypeStruct((n, 64), F32)),
        scratch_types=[
            pltpu.VMEM_SHARED((n, 64), F32),
            pltpu.VMEM((zrows, 64), F32),
            pltpu.VMEM((KI, 64), F32), pltpu.VMEM((KI, 64), F32),
            pltpu.VMEM((KI,), I32), pltpu.VMEM((KI,), I32),
            pltpu.SemaphoreType.DMA, pltpu.SemaphoreType.DMA,
            pltpu.SemaphoreType.DMA, pltpu.SemaphoreType.DMA,
            pltpu.SemaphoreType.DMA, pltpu.SemaphoreType.DMA,
        ],
    )
    def k(elo_r, ehi_r, do_r, nnlo_r, nnhi_r, acc, zbuf,
          ebuf0, ebuf1, idxb0, idxb1, se0, se1, si0, si1, sc0, sc1):
        c = lax.axis_index("c")
        s = lax.axis_index("s")
        ebuf = (ebuf0, ebuf1)
        idxb = (idxb0, idxb1)
        se = (se0, se1)
        si = (si0, si1)
        sc = (sc0, sc1)

        @pl.loop(0, zrows)
        def _(r):
            for cc in range(4):
                zbuf[r, pl.ds(cc * 16, 16)] = jnp.zeros((16,), F32)

        @pl.loop(0, rows_per_tile // zrows)
        def _(j):
            pltpu.sync_copy(zbuf, acc.at[pl.ds(s * rows_per_tile + j * zrows,
                                               zrows)])
        plsc.subcore_barrier()

        def phase(src_hbm, out_hbm):
            def fire(r0, b):
                ce = pltpu.async_copy(src_hbm.at[pl.ds(r0, KI)], ebuf[b], se[b])
                ci = pltpu.async_copy(do_r.at[pl.ds(r0, KI)], idxb[b], si[b])
                return ce, ci

            def process(b, ce, ci):
                ce.wait()
                ci.wait()
                return pltpu.async_copy(ebuf[b], acc.at[idxb[b]], sc[b],
                                        add=True)

            @pl.loop(0, nit - 1, step=2)
            def _(i):
                r0 = s * et + i * KI
                d0 = fire(r0, 0)
                d1 = fire(r0 + KI, 1)
                w0 = process(0, *d0)
                w1 = process(1, *d1)
                w0.wait()
                w1.wait()

            if nit % 2:
                r0 = s * et + (nit - 1) * KI
                d0 = fire(r0, 0)
                process(0, *d0).wait()

            plsc.subcore_barrier()
            r0 = s * rows_per_tile
            pltpu.sync_copy(acc.at[pl.ds(r0, rows_per_tile)],
                            out_hbm.at[pl.ds(r0, rows_per_tile)])

        @pl.when(c == 0)
        def _():
            phase(elo_r, nnlo_r)

        @pl.when(c == 1)
        def _():
            phase(ehi_r, nnhi_r)

    return k(efi_lo, efi_hi, dst_o)


def _sc_seg_f1f2(f12, dst_i, pos, n, p):
    e = f12.shape[0]
    et = e // NS
    nit = et // KI
    half = p // NC
    sum_rows = 2560
    cmb_stride, cmb_len = 156, 160
    neg = jnp.float32(-jnp.inf)
    mesh = plsc.VectorSubcoreMesh(core_axis_name="c", subcore_axis_name="s", num_cores=NC, num_subcores=NS)

    @functools.partial(
        pl.kernel, mesh=mesh,
        compiler_params=pltpu.CompilerParams(use_tc_tiling_on_sc=False, needs_layout_passes=False),
        out_type=(jax.ShapeDtypeStruct((p, 32), F32),
                  jax.ShapeDtypeStruct((p, 32), F32)),
        scratch_types=[
            pltpu.VMEM((half + 8, 32), F32),
            pltpu.VMEM((n,), I32),
            pltpu.VMEM_SHARED((sum_rows, 32), F32),
            pltpu.VMEM_SHARED((NS, 320, 32), F32),
            pltpu.VMEM((KI, 32), F32), pltpu.VMEM((KI, 32), F32),
            pltpu.VMEM((KI, 32), F32), pltpu.VMEM((KI, 32), F32),
            pltpu.VMEM((KI,), I32), pltpu.VMEM((KI,), I32),
            pltpu.VMEM((KI,), I32), pltpu.VMEM((KI,), I32),
            pltpu.VMEM((20, 32), F32), pltpu.VMEM((20, 32), F32),
            pltpu.SemaphoreType.DMA, pltpu.SemaphoreType.DMA,
            pltpu.SemaphoreType.DMA, pltpu.SemaphoreType.DMA,
            pltpu.SemaphoreType.DMA, pltpu.SemaphoreType.DMA,
            pltpu.SemaphoreType.DMA, pltpu.SemaphoreType.DMA,
        ],
    )
    def k(f12_r, di_r, pos_r, nfo1_r, nfo2_r,
          macc, posb, sacc, stage, f1b0, f1b1, f2b0, f2b1,
          db0, db1, ib0, ib1, cmb, tbuf,
          sf0, sf1, sg0, sg1, sd0, sd1, sc0, sc1):
        c = lax.axis_index("c")
        s = lax.axis_index("s")
        base = c * half
        f1b = (f1b0, f1b1)
        f2b = (f2b0, f2b1)
        db = (db0, db1)
        ib = (ib0, ib1)
        sf = (sf0, sf1)
        sg = (sg0, sg1)
        sd = (sd0, sd1)
        sc_ = (sc0, sc1)

        pltpu.sync_copy(pos_r, posb)

        @pl.loop(0, half)
        def _(r):
            macc[r, pl.ds(0, 16)] = jnp.full((16,), neg, F32)
            macc[r, pl.ds(16, 16)] = jnp.full((16,), neg, F32)

        @pl.loop(0, KI)
        def _(r):
            for cc in range(2):
                f1b0[r, pl.ds(cc * 16, 16)] = jnp.zeros((16,), F32)

        pltpu.sync_copy(f1b0, sacc.at[pl.ds(s * 160, KI)])
        pltpu.sync_copy(f1b0, sacc.at[pl.ds(s * 160 + KI, KI)])
        plsc.subcore_barrier()

        def fire(r0, b):
            cf = pltpu.async_copy(f12_r.at[pl.ds(r0, KI), pl.ds(0, 32)],
                                  f1b[b], sf[b])
            cg = pltpu.async_copy(f12_r.at[pl.ds(r0, KI), pl.ds(32, 32)],
                                  f2b[b], sg[b])
            cd = pltpu.async_copy(di_r.at[pl.ds(r0, KI)], db[b], sd[b])
            return cf, cg, cd

        def process(b, cf, cg, cd):
            cd.wait()

            @pl.loop(0, KI // 16)
            def _(j):
                sl = pl.ds(j * 16, 16)
                lv = plsc.load_gather(posb, [db[b][sl]]) - base
                valid = (lv >= 0) & (lv < half)
                ib[b][sl] = jnp.where(valid, lv, half)

            cf.wait()
            w = pltpu.async_copy(f1b[b], sacc.at[ib[b]], sc_[b], add=True)
            cg.wait()

            @pl.loop(0, KI // 16)
            def _(j):
                idx16 = ib[b][pl.ds(j * 16, 16)]
                for l in range(16):
                    lv = idx16[l]
                    for cc in range(2):
                        sl = pl.ds(cc * 16, 16)
                        macc[lv, sl] = jnp.maximum(
                            macc[lv, sl], f2b[b][j * 16 + l, sl])

            return w

        @pl.loop(0, nit - 1, step=2)
        def _(i):
            r0 = s * et + i * KI
            d0 = fire(r0, 0)
            d1 = fire(r0 + KI, 1)
            w0 = process(0, *d0)
            w1 = process(1, *d1)
            w0.wait()
            w1.wait()

        if nit % 2:
            r0 = s * et + (nit - 1) * KI
            d0 = fire(r0, 0)
            process(0, *d0).wait()

        plsc.subcore_barrier()

        for rnd in range(8):
            rbase = rnd * 312
            rows_limit = min(320, half - rbase)
            pltpu.sync_copy(macc.at[pl.ds(rbase, 320)], stage.at[s])
            plsc.subcore_barrier()
            l0 = jnp.minimum(s * 20, rows_limit - 20)
            pltpu.sync_copy(stage.at[0, pl.ds(l0, 20)], cmb)

            @pl.loop(1, NS)
            def _(t):
                pltpu.sync_copy(stage.at[t, pl.ds(l0, 20)], tbuf)

                @pl.loop(0, 20)
                def _(r):
                    for cc in range(2):
                        sl = pl.ds(cc * 16, 16)
                        cmb[r, sl] = jnp.maximum(cmb[r, sl], tbuf[r, sl])

            @pl.loop(0, 20)
            def _(r):
                for cc in range(2):
                    sl = pl.ds(cc * 16, 16)
                    v = cmb[r, sl]
                    cmb[r, sl] = jnp.where(v == neg, jnp.zeros((16,), F32), v)

            pltpu.sync_copy(cmb, nfo2_r.at[pl.ds(base + rbase + l0, 20)])
            plsc.subcore_barrier()

        r0s = s * cmb_stride
        pltpu.sync_copy(sacc.at[pl.ds(r0s, cmb_len)],
                        nfo1_r.at[pl.ds(base + r0s, cmb_len)])

    return k(f12, dst_i, pos)


def _sc_assemble(nn_lo, nn_hi, new_val, onodes):
    n = nn_lo.shape[0]
    p = new_val.shape[0]
    rows_per_tile = n // NS
    slot_stride = 312

    mesh = plsc.VectorSubcoreMesh(core_axis_name="c", subcore_axis_name="s", num_cores=NC, num_subcores=NS)

    @functools.partial(
        pl.kernel, mesh=mesh,
        compiler_params=pltpu.CompilerParams(use_tc_tiling_on_sc=False, needs_layout_passes=False),
        out_type=jax.ShapeDtypeStruct((n, 128), F32),
        scratch_types=[
            pltpu.VMEM((rows_per_tile, 64), F32),
            pltpu.VMEM((KI,), I32),
            pltpu.VMEM((KI, 128), F32),
        ],
    )
    def k(lo_r, hi_r, nv_r, on_r, out_r, bounce, idxb, vbuf):
        c = lax.axis_index("c")
        s = lax.axis_index("s")

        @pl.when(c == 0)
        def _():
            r0 = s * rows_per_tile
            pltpu.sync_copy(lo_r.at[pl.ds(r0, rows_per_tile)], bounce)
            pltpu.sync_copy(bounce, out_r.at[pl.ds(r0, rows_per_tile),
                                             pl.ds(0, 64)])
            pltpu.sync_copy(hi_r.at[pl.ds(r0, rows_per_tile)], bounce)
            pltpu.sync_copy(bounce, out_r.at[pl.ds(r0, rows_per_tile),
                                             pl.ds(64, 64)])
            plsc.subcore_barrier()

            @pl.loop(0, 4)
            def _(j):
                off = s * slot_stride + j * KI
                pltpu.sync_copy(on_r.at[pl.ds(off, KI)], idxb)
                pltpu.sync_copy(nv_r.at[pl.ds(off, KI)], vbuf)
                pltpu.sync_copy(vbuf, out_r.at[idxb])

    return k(nn_lo, nn_hi, new_val, onodes)


def kernel(nf, ef_out, ef_in, params_msg_i2o, params_reduce_o, params_msg_o2i,
           edge_index_out, edge_index_in, output_nodes):
    n = nf.shape[0]
    p = output_nodes.shape[0]

    (w1, b1), (w2, b2), (w3, b3), (w4, b4), (w5, b5) = params_msg_o2i
    (v1, c1), (v2, c2), (v3, c3), (v4, c4) = params_msg_i2o
    (u1, d1), (u2, d2), (u3, d3), (u4, d4) = params_reduce_o

    wcat = jnp.concatenate(
        [w1[:128], w1[128:256], v1[:128], v1[128:256], u1[:128]], axis=1)
    w1c = w1[256:272]
    v1c = v1[256:272]
    u1b = u1[128:160]
    u1c = u1[160:192]
    v4p = jnp.zeros((64, 128), F32)
    v4p = v4p.at[:, 0:32].set(v4[:, 1:33])
    v4p = v4p.at[:, 32:64].set(v4[:, 33:65])
    v4p = v4p.at[:, 64:65].set(v4[:, 0:1])
    c4p = jnp.zeros((1, 128), F32)
    c4p = c4p.at[0, 0:32].set(c4[1:33])
    c4p = c4p.at[0, 32:64].set(c4[33:65])
    c4p = c4p.at[0, 64].set(c4[0])

    row = lambda x: x.reshape(1, -1)

    src_o, dst_o = edge_index_out[0], edge_index_out[1]
    src_i, dst_i = edge_index_in[0], edge_index_in[1]

    a2, b2v, a1, b1v, dproj = _tc_precompute(nf, wcat)

    h0_o, h0_i, dout, pos = _sc_gather(a2, b2v, a1, b1v, dproj,
                                       src_o, dst_o, src_i, dst_i,
                                       output_nodes, n)

    efi_lo, efi_hi, f12 = _tc_mlp_edges(
        h0_o, ef_out, h0_i, ef_in,
        w1c, row(b1), w2, row(b2), w3, row(b3), w4, row(b4), w5, row(b5),
        v1c, row(c1), v2, row(c2), v3, row(c3), v4p, c4p)

    nn_lo, nn_hi = _sc_segsum_efi(efi_lo, efi_hi, dst_o, n)

    nfo1, nfo2 = _sc_seg_f1f2(f12, dst_i, pos, n, p)

    new_val = _tc_reduce_o(dout, nfo1, nfo2, u1b, u1c, row(d1), u2, row(d2),
                           u3, row(d3), u4, row(d4))

    return _sc_assemble(nn_lo, nn_hi, new_val, output_nodes)
